# baseline (speedup 1.0000x reference)
import functools

import jax
import jax.numpy as jnp
from jax import lax
from jax.experimental import pallas as pl
from jax.experimental.pallas import tpu as pltpu
from jax.experimental.pallas import tpu_sc as plsc

B, S, D, E, K = 4, 2048, 768, 8, 2
T = B * S
TM = 256
EPAD = 128
NT = T // TM
TPAD = 2 * T + E * TM
MT = TPAD // TM
NC, NS = 2, 16
NW = NC * NS
TPW = T // NW
CH = 64


def _router_kernel(x_ref, wg_ref, bg_ref, meta_ref, offs_ref, te_ref, carry):
    m = pl.program_id(0)
    x = x_ref[...]
    logits = jnp.dot(x, wg_ref[...], preferred_element_type=jnp.float32)
    logits = logits + bg_ref[0][None, :]
    mx = jnp.max(logits, axis=-1, keepdims=True)
    ex = jnp.exp(logits - mx)
    p = ex / jnp.sum(ex, axis=-1, keepdims=True)
    ii = lax.broadcasted_iota(jnp.int32, (TM, EPAD), 1)
    m0 = jnp.max(p, axis=-1, keepdims=True)
    e0 = jnp.min(jnp.where(p == m0, ii, EPAD), axis=-1, keepdims=True)
    sel0 = ii == e0
    p1 = jnp.where(sel0, -1.0, p)
    m1 = jnp.max(p1, axis=-1, keepdims=True)
    e1 = jnp.min(jnp.where(p1 == m1, ii, EPAD), axis=-1, keepdims=True)
    sel1 = ii == e1

    mask_f = jnp.where(sel0 | sel1, 1.0, 0.0)
    ti = lax.broadcasted_iota(jnp.int32, (TM, TM), 0)
    tj = lax.broadcasted_iota(jnp.int32, (TM, TM), 1)
    ltri = jnp.where(ti > tj, 1.0, 0.0)
    rex = jnp.dot(ltri, mask_f, preferred_element_type=jnp.float32)
    counts = jnp.sum(mask_f, axis=0, keepdims=True)

    @pl.when(m == 0)
    def _():
        carry[...] = jnp.zeros_like(carry)

    rank_g = rex + carry[0:1, :]
    carry[0:1, :] = carry[0:1, :] + counts

    r0 = jnp.sum(jnp.where(sel0, rank_g, 0.0), axis=-1, keepdims=True)
    r1 = jnp.sum(jnp.where(sel1, rank_g, 0.0), axis=-1, keepdims=True)
    meta = (jnp.where(ii == 0, e0.astype(jnp.float32), 0.0)
            + jnp.where(ii == 1, e1.astype(jnp.float32), 0.0)
            + jnp.where(ii == 2, r0, 0.0)
            + jnp.where(ii == 3, r1, 0.0)
            + jnp.where(ii == 4, m0, 0.0)
            + jnp.where(ii == 5, m1, 0.0))
    meta_ref[...] = meta

    @pl.when(m == NT - 1)
    def _():
        total = carry[0:1, :]
        padded = jnp.floor((total + (TM - 1)) / TM) * TM
        ei = lax.broadcasted_iota(jnp.int32, (EPAD, EPAD), 0)
        ej = lax.broadcasted_iota(jnp.int32, (EPAD, EPAD), 1)
        lt = jnp.where(ei < ej, 1.0, 0.0)
        offs = jnp.dot(padded, lt, preferred_element_type=jnp.float32)
        offs_ref[...] = offs
        ends = offs + padded
        m256 = (lax.broadcasted_iota(jnp.int32, (EPAD, EPAD), 0)
                * TM).astype(jnp.float32)
        live_e = lax.broadcasted_iota(jnp.int32, (EPAD, EPAD), 1) < E
        ge = jnp.where((m256 >= ends) & live_e, 1.0, 0.0)
        te = jnp.sum(ge, axis=-1, keepdims=True)
        te_ref[...] = jnp.broadcast_to(te, (EPAD, EPAD)).astype(jnp.int32)


def _pos_group(metab, offsv, g):
    rowbase = (lax.iota(jnp.int32, 16) + g * 16) * EPAD
    e0 = plsc.load_gather(metab, [rowbase]).astype(jnp.int32)
    e1 = plsc.load_gather(metab, [rowbase + 1]).astype(jnp.int32)
    r0 = plsc.load_gather(metab, [rowbase + 2])
    r1 = plsc.load_gather(metab, [rowbase + 3])
    p0 = (plsc.load_gather(offsv, [e0]) + r0).astype(jnp.int32)
    p1 = (plsc.load_gather(offsv, [e1]) + r1).astype(jnp.int32)
    return p0, p1


def _dispatch_body(x_hbm, meta_hbm, offs_hbm, xs_hbm,
                   metab, xbuf, pos0b, pos1b, offsv, sem):
    wid = lax.axis_index("s") * NC + lax.axis_index("c")
    pltpu.sync_copy(offs_hbm.at[0, pl.ds(0, 16)], offsv)
    base = wid * TPW
    for ci in range(TPW // CH):
        cb = base + ci * CH
        pltpu.sync_copy(meta_hbm.at[pl.ds(cb * EPAD, CH * EPAD)], metab)
        pltpu.sync_copy(x_hbm.at[pl.ds(cb, CH)], xbuf)
        for g in range(CH // 16):
            p0, p1 = _pos_group(metab, offsv, g)
            pos0b[pl.ds(g * 16, 16)] = p0
            pos1b[pl.ds(g * 16, 16)] = p1
        c0 = pltpu.async_copy(xbuf, xs_hbm.at[pos0b], sem)
        c1 = pltpu.async_copy(xbuf, xs_hbm.at[pos1b], sem)
        c0.wait()
        c1.wait()


def _mm_kernel(te_ref, xs_ref, we_ref, be_ref, ys_ref):
    m = pl.program_id(0)

    @pl.when(te_ref[m] < E)
    def _():
        acc = jnp.dot(xs_ref[...].astype(jnp.bfloat16), we_ref[0],
                      preferred_element_type=jnp.float32)
        ys_ref[...] = acc + be_ref[0, 0][None, :]


def _combine_body(ys_hbm, meta_hbm, offs_hbm, out_hbm,
                  metab, y0buf, y1buf, pos0b, pos1b, offsv, sem):
    wid = lax.axis_index("s") * NC + lax.axis_index("c")
    pltpu.sync_copy(offs_hbm.at[0, pl.ds(0, 16)], offsv)
    base = wid * TPW
    for ci in range(TPW // CH):
        cb = base + ci * CH
        pltpu.sync_copy(meta_hbm.at[pl.ds(cb * EPAD, CH * EPAD)], metab)
        for g in range(CH // 16):
            p0, p1 = _pos_group(metab, offsv, g)
            pos0b[pl.ds(g * 16, 16)] = p0
            pos1b[pl.ds(g * 16, 16)] = p1
        c0 = pltpu.async_copy(ys_hbm.at[pos0b], y0buf, sem)
        c1 = pltpu.async_copy(ys_hbm.at[pos1b], y1buf, sem)
        c0.wait()
        c1.wait()

        def tok(i, _):
            rb = (jnp.zeros((16,), jnp.int32) + i) * EPAD
            w0 = plsc.load_gather(metab, [rb + 4])
            w1 = plsc.load_gather(metab, [rb + 5])
            for j in range(D // 16):
                sl = pl.ds(j * 16, 16)
                y0buf[i, sl] = w0 * y0buf[i, sl] + w1 * y1buf[i, sl]
            return 0

        lax.fori_loop(0, CH, tok, 0)
        pltpu.sync_copy(y0buf, out_hbm.at[pl.ds(cb, CH)])


@functools.lru_cache(maxsize=1)
def _sc_kernels():
    mesh = plsc.VectorSubcoreMesh(core_axis_name="c", subcore_axis_name="s")
    params = pltpu.CompilerParams(needs_layout_passes=False)
    dispatch = pl.kernel(
        _dispatch_body, mesh=mesh, compiler_params=params,
        out_type=jax.ShapeDtypeStruct((TPAD, D), jnp.float32),
        scratch_types=[
            pltpu.VMEM((CH * EPAD,), jnp.float32),
            pltpu.VMEM((CH, D), jnp.float32),
            pltpu.VMEM((CH,), jnp.int32),
            pltpu.VMEM((CH,), jnp.int32),
            pltpu.VMEM((16,), jnp.float32),
            pltpu.SemaphoreType.DMA,
        ])
    combine = pl.kernel(
        _combine_body, mesh=mesh, compiler_params=params,
        out_type=jax.ShapeDtypeStruct((T, D), jnp.float32),
        scratch_types=[
            pltpu.VMEM((CH * EPAD,), jnp.float32),
            pltpu.VMEM((CH, D), jnp.float32),
            pltpu.VMEM((CH, D), jnp.float32),
            pltpu.VMEM((CH,), jnp.int32),
            pltpu.VMEM((CH,), jnp.int32),
            pltpu.VMEM((16,), jnp.float32),
            pltpu.SemaphoreType.DMA,
        ])
    return dispatch, combine


def kernel(input_tensor, Wg, bg, We, be):
    x = input_tensor.reshape(T, D)
    wg = jnp.pad(Wg, ((0, 0), (0, EPAD - E)))
    bgp = jnp.pad(bg, (0, EPAD - E), constant_values=-1e30).reshape(1, EPAD)
    we_bf = We.astype(jnp.bfloat16)

    meta, offs, te_full = pl.pallas_call(
        _router_kernel,
        grid=(NT,),
        in_specs=[
            pl.BlockSpec((TM, D), lambda m: (m, 0)),
            pl.BlockSpec((D, EPAD), lambda m: (0, 0)),
            pl.BlockSpec((1, EPAD), lambda m: (0, 0)),
        ],
        out_specs=[
            pl.BlockSpec((TM, EPAD), lambda m: (m, 0)),
            pl.BlockSpec((1, EPAD), lambda m: (0, 0)),
            pl.BlockSpec((EPAD, EPAD), lambda m: (0, 0)),
        ],
        out_shape=[
            jax.ShapeDtypeStruct((T, EPAD), jnp.float32),
            jax.ShapeDtypeStruct((1, EPAD), jnp.float32),
            jax.ShapeDtypeStruct((EPAD, EPAD), jnp.int32),
        ],
        scratch_shapes=[pltpu.VMEM((8, EPAD), jnp.float32)],
    )(x, wg, bgp)
    te = te_full[:, 0]

    dispatch, combine = _sc_kernels()
    meta_flat = meta.reshape(T * EPAD)
    xs = dispatch(x, meta_flat, offs)

    ys = pl.pallas_call(
        _mm_kernel,
        grid_spec=pltpu.PrefetchScalarGridSpec(
            num_scalar_prefetch=1,
            grid=(MT,),
            in_specs=[
                pl.BlockSpec((TM, D), lambda m, te_r: (m, 0)),
                pl.BlockSpec((1, D, D),
                             lambda m, te_r: (jnp.minimum(te_r[m], E - 1), 0, 0)),
                pl.BlockSpec((1, 1, D),
                             lambda m, te_r: (jnp.minimum(te_r[m], E - 1), 0, 0)),
            ],
            out_specs=pl.BlockSpec((TM, D), lambda m, te_r: (m, 0)),
        ),
        out_shape=jax.ShapeDtypeStruct((TPAD, D), jnp.float32),
    )(te, xs, we_bf, be.reshape(E, 1, D))

    out = combine(ys, meta_flat, offs)
    return out.reshape(B, S, D)


# speedup vs baseline: 1.0710x; 1.0710x over previous
import functools

import jax
import jax.numpy as jnp
from jax import lax
from jax.experimental import pallas as pl
from jax.experimental.pallas import tpu as pltpu
from jax.experimental.pallas import tpu_sc as plsc

B, S, D, E, K = 4, 2048, 768, 8, 2
T = B * S
TR = 512
TM = 256
EPAD = 128
NR = T // TR
TPAD = 2 * T + E * TM
MT = TPAD // TM
NC, NS = 2, 16
NW = NC * NS
TPW = T // NW
CH = 64


def _router_kernel(x_ref, wg_ref, bg_ref, meta_ref, offs_ref, te_ref, carry):
    m = pl.program_id(0)
    x = x_ref[...]
    logits = jnp.dot(x, wg_ref[...], preferred_element_type=jnp.float32)
    logits = logits + bg_ref[0][None, :]
    mx = jnp.max(logits, axis=-1, keepdims=True)
    ex = jnp.exp(logits - mx)
    z = jnp.sum(ex, axis=-1, keepdims=True)
    w0 = 1.0 / z
    ii = lax.broadcasted_iota(jnp.int32, (TR, EPAD), 1)
    e0 = jnp.min(jnp.where(logits == mx, ii, EPAD), axis=-1, keepdims=True)
    sel0 = ii == e0
    l1 = jnp.max(jnp.where(sel0, -jnp.inf, logits), axis=-1, keepdims=True)
    w1 = jnp.exp(l1 - mx) / z
    e1 = jnp.min(jnp.where((logits == l1) & (~sel0), ii, EPAD),
                 axis=-1, keepdims=True)
    sel1 = ii == e1

    mask_f = jnp.where(sel0 | sel1, 1.0, 0.0)
    ti = lax.broadcasted_iota(jnp.int32, (TR, TR), 0)
    tj = lax.broadcasted_iota(jnp.int32, (TR, TR), 1)
    ltri = jnp.where(ti > tj, 1.0, 0.0).astype(jnp.bfloat16)
    rex = jnp.dot(ltri, mask_f.astype(jnp.bfloat16),
                  preferred_element_type=jnp.float32)
    counts = jnp.sum(mask_f, axis=0, keepdims=True)

    @pl.when(m == 0)
    def _():
        carry[...] = jnp.zeros_like(carry)

    rank_g = rex + carry[0:1, :]
    carry[0:1, :] = carry[0:1, :] + counts

    r0 = jnp.sum(jnp.where(sel0, rank_g, 0.0), axis=-1, keepdims=True)
    r1 = jnp.sum(jnp.where(sel1, rank_g, 0.0), axis=-1, keepdims=True)
    meta_ref[...] = (jnp.where(ii == 0, e0.astype(jnp.float32), 0.0)
                     + jnp.where(ii == 1, e1.astype(jnp.float32), 0.0)
                     + jnp.where(ii == 2, r0, 0.0)
                     + jnp.where(ii == 3, r1, 0.0)
                     + jnp.where(ii == 4, w0, 0.0)
                     + jnp.where(ii == 5, w1, 0.0))

    @pl.when(m == NR - 1)
    def _():
        total = carry[0:1, :]
        padded = jnp.floor((total + (TM - 1)) / TM) * TM
        ei = lax.broadcasted_iota(jnp.int32, (EPAD, EPAD), 0)
        ej = lax.broadcasted_iota(jnp.int32, (EPAD, EPAD), 1)
        lt = jnp.where(ei < ej, 1.0, 0.0)
        offs = jnp.dot(padded, lt, preferred_element_type=jnp.float32)
        offs_ref[...] = offs
        ends = offs + padded
        m256 = (lax.broadcasted_iota(jnp.int32, (EPAD, EPAD), 0)
                * TM).astype(jnp.float32)
        live_e = lax.broadcasted_iota(jnp.int32, (EPAD, EPAD), 1) < E
        ge = jnp.where((m256 >= ends) & live_e, 1.0, 0.0)
        te = jnp.sum(ge, axis=-1, keepdims=True)
        te_ref[...] = jnp.broadcast_to(te, (EPAD, EPAD)).astype(jnp.int32)


def _pos_group(metab, offsv, g):
    rowbase = (lax.iota(jnp.int32, 16) + g * 16) * EPAD
    e0 = plsc.load_gather(metab, [rowbase]).astype(jnp.int32)
    e1 = plsc.load_gather(metab, [rowbase + 1]).astype(jnp.int32)
    r0 = plsc.load_gather(metab, [rowbase + 2])
    r1 = plsc.load_gather(metab, [rowbase + 3])
    p0 = (plsc.load_gather(offsv, [e0]) + r0).astype(jnp.int32)
    p1 = (plsc.load_gather(offsv, [e1]) + r1).astype(jnp.int32)
    return p0, p1


def _dispatch_body(x_hbm, meta_hbm, offs_hbm, xs_hbm,
                   metab, xbuf, pos0b, pos1b, offsv, sem):
    wid = lax.axis_index("s") * NC + lax.axis_index("c")
    pltpu.sync_copy(offs_hbm.at[0, pl.ds(0, 16)], offsv)
    base = wid * TPW
    for ci in range(TPW // CH):
        cb = base + ci * CH
        pltpu.sync_copy(meta_hbm.at[pl.ds(cb * EPAD, CH * EPAD)], metab)
        pltpu.sync_copy(x_hbm.at[pl.ds(cb, CH)], xbuf)
        for g in range(CH // 16):
            p0, p1 = _pos_group(metab, offsv, g)
            pos0b[pl.ds(g * 16, 16)] = p0
            pos1b[pl.ds(g * 16, 16)] = p1
        c0 = pltpu.async_copy(xbuf, xs_hbm.at[pos0b], sem)
        c1 = pltpu.async_copy(xbuf, xs_hbm.at[pos1b], sem)
        c0.wait()
        c1.wait()


def _mm_kernel(te_ref, xs_ref, we_ref, be_ref, ys_ref):
    m = pl.program_id(0)

    @pl.when(te_ref[m] < E)
    def _():
        acc = jnp.dot(xs_ref[...].astype(jnp.bfloat16), we_ref[0],
                      preferred_element_type=jnp.float32)
        ys_ref[...] = acc + be_ref[0, 0][None, :]


def _combine_body(ys_hbm, meta_hbm, offs_hbm, out_hbm,
                  metab, y0buf, y1buf, pos0b, pos1b, offsv, sem):
    wid = lax.axis_index("s") * NC + lax.axis_index("c")
    pltpu.sync_copy(offs_hbm.at[0, pl.ds(0, 16)], offsv)
    base = wid * TPW
    for ci in range(TPW // CH):
        cb = base + ci * CH
        pltpu.sync_copy(meta_hbm.at[pl.ds(cb * EPAD, CH * EPAD)], metab)
        for g in range(CH // 16):
            p0, p1 = _pos_group(metab, offsv, g)
            pos0b[pl.ds(g * 16, 16)] = p0
            pos1b[pl.ds(g * 16, 16)] = p1
        c0 = pltpu.async_copy(ys_hbm.at[pos0b], y0buf, sem)
        c1 = pltpu.async_copy(ys_hbm.at[pos1b], y1buf, sem)
        c0.wait()
        c1.wait()

        def tok(i, _):
            rb = (jnp.zeros((16,), jnp.int32) + i) * EPAD
            w0 = plsc.load_gather(metab, [rb + 4])
            w1 = plsc.load_gather(metab, [rb + 5])
            for j in range(D // 16):
                sl = pl.ds(j * 16, 16)
                y0buf[i, sl] = w0 * y0buf[i, sl] + w1 * y1buf[i, sl]
            return 0

        lax.fori_loop(0, CH, tok, 0)
        pltpu.sync_copy(y0buf, out_hbm.at[pl.ds(cb, CH)])


@functools.lru_cache(maxsize=1)
def _sc_kernels():
    mesh = plsc.VectorSubcoreMesh(core_axis_name="c", subcore_axis_name="s")
    params = pltpu.CompilerParams(needs_layout_passes=False)
    dispatch = pl.kernel(
        _dispatch_body, mesh=mesh, compiler_params=params,
        out_type=jax.ShapeDtypeStruct((TPAD, D), jnp.float32),
        scratch_types=[
            pltpu.VMEM((CH * EPAD,), jnp.float32),
            pltpu.VMEM((CH, D), jnp.float32),
            pltpu.VMEM((CH,), jnp.int32),
            pltpu.VMEM((CH,), jnp.int32),
            pltpu.VMEM((16,), jnp.float32),
            pltpu.SemaphoreType.DMA,
        ])
    combine = pl.kernel(
        _combine_body, mesh=mesh, compiler_params=params,
        out_type=jax.ShapeDtypeStruct((T, D), jnp.float32),
        scratch_types=[
            pltpu.VMEM((CH * EPAD,), jnp.float32),
            pltpu.VMEM((CH, D), jnp.float32),
            pltpu.VMEM((CH, D), jnp.float32),
            pltpu.VMEM((CH,), jnp.int32),
            pltpu.VMEM((CH,), jnp.int32),
            pltpu.VMEM((16,), jnp.float32),
            pltpu.SemaphoreType.DMA,
        ])
    return dispatch, combine


def kernel(input_tensor, Wg, bg, We, be):
    x = input_tensor.reshape(T, D)
    wg = jnp.pad(Wg, ((0, 0), (0, EPAD - E)))
    bgp = jnp.pad(bg, (0, EPAD - E), constant_values=-1e30).reshape(1, EPAD)
    we_bf = We.astype(jnp.bfloat16)

    meta, offs, te_full = pl.pallas_call(
        _router_kernel,
        grid=(NR,),
        in_specs=[
            pl.BlockSpec((TR, D), lambda m: (m, 0)),
            pl.BlockSpec((D, EPAD), lambda m: (0, 0)),
            pl.BlockSpec((1, EPAD), lambda m: (0, 0)),
        ],
        out_specs=[
            pl.BlockSpec((TR, EPAD), lambda m: (m, 0)),
            pl.BlockSpec((1, EPAD), lambda m: (0, 0)),
            pl.BlockSpec((EPAD, EPAD), lambda m: (0, 0)),
        ],
        out_shape=[
            jax.ShapeDtypeStruct((T, EPAD), jnp.float32),
            jax.ShapeDtypeStruct((1, EPAD), jnp.float32),
            jax.ShapeDtypeStruct((EPAD, EPAD), jnp.int32),
        ],
        scratch_shapes=[pltpu.VMEM((8, EPAD), jnp.float32)],
    )(x, wg, bgp)
    te = te_full[:, 0]

    dispatch, combine = _sc_kernels()
    meta_flat = meta.reshape(T * EPAD)
    xs = dispatch(x, meta_flat, offs)

    ys = pl.pallas_call(
        _mm_kernel,
        grid_spec=pltpu.PrefetchScalarGridSpec(
            num_scalar_prefetch=1,
            grid=(MT,),
            in_specs=[
                pl.BlockSpec((TM, D), lambda m, te_r: (m, 0)),
                pl.BlockSpec((1, D, D),
                             lambda m, te_r: (jnp.minimum(te_r[m], E - 1), 0, 0)),
                pl.BlockSpec((1, 1, D),
                             lambda m, te_r: (jnp.minimum(te_r[m], E - 1), 0, 0)),
            ],
            out_specs=pl.BlockSpec((TM, D), lambda m, te_r: (m, 0)),
        ),
        out_shape=jax.ShapeDtypeStruct((TPAD, D), jnp.float32),
    )(te, xs, we_bf, be.reshape(E, 1, D))

    out = combine(ys, meta_flat, offs)
    return out.reshape(B, S, D)
